# baseline (speedup 1.0000x reference)
import functools

import numpy as np
import jax
import jax.numpy as jnp
from jax import lax
from jax.experimental import pallas as pl
from jax.experimental.pallas import tpu as pltpu
from jax.experimental.pallas import tpu_sc as plsc

LN_EPS = 1e-3
L = 16
SPC = 4
BB = 128


def _posenc_np(max_len, d):
    pos = np.arange(max_len)[:, None]
    i = np.arange(d)[None, :]
    ang = pos * (1.0 / np.power(10000, 2 * (i // 2) / np.float32(d)))
    ang[:, 0::2] = np.sin(ang[:, 0::2])
    ang[:, 1::2] = np.cos(ang[:, 1::2])
    return ang.astype(np.float32)


def _rsqrt_vec(v):
    i = lax.bitcast_convert_type(v, jnp.int32)
    i = jnp.int32(0x5F3759DF) - lax.shift_right_logical(i, jnp.int32(1))
    y = lax.bitcast_convert_type(i, jnp.float32)
    for _ in range(2):
        y = y * (1.5 - 0.5 * v * y * y)
    return y


@functools.lru_cache(maxsize=None)
def _build(B, S, E):
    info = plsc.get_sparse_core_info()
    NC, NS = info.num_cores, info.num_subcores
    NW = NC * NS
    assert B % (NW * BB) == 0 and B // NW == BB
    CHUNK = SPC * BB
    NCHUNK = S // SPC
    assert NCHUNK % 2 == 0 and S % SPC == 0
    NGS = BB // L
    mesh = plsc.VectorSubcoreMesh(core_axis_name="c", subcore_axis_name="s")

    @functools.partial(
        pl.kernel,
        mesh=mesh,
        compiler_params=pltpu.CompilerParams(
            use_tc_tiling_on_sc=False, needs_layout_passes=False),
        out_type=jax.ShapeDtypeStruct((S, E // 8, B // BB, 8, BB),
                                      jnp.float32),
        scratch_types=[
            pltpu.VMEM((2, CHUNK), jnp.int32),
            pltpu.VMEM((CHUNK, E), jnp.float32),
            pltpu.VMEM((CHUNK, E), jnp.float32),
            pltpu.VMEM((E // 8, 8, BB), jnp.float32),
            pltpu.VMEM((E // 8, 8, BB), jnp.float32),
            pltpu.VMEM((S, E + L), jnp.float32),
            pltpu.VMEM((2, E, L), jnp.float32),
            [pltpu.SemaphoreType.DMA] * 6,
        ],
    )
    def k(idsT_hbm, table_hbm, posf_hbm, gbT_hbm, out_hbm,
          ids_v, rows0, rows1, xt0, xt1, posf_v, gbT_v, sems):
        cid = lax.axis_index("c")
        sid = lax.axis_index("s")
        wid = sid * NC + cid
        bw = wid * BB
        rows = [rows0, rows1]
        xts = [xt0, xt1]
        sem_ids = [sems[0], sems[1]]
        sem_g = [sems[2], sems[3]]
        sem_x = [sems[4], sems[5]]

        pltpu.sync_copy(posf_hbm, posf_v)
        pltpu.sync_copy(gbT_hbm, gbT_v)
        lanes = lax.iota(jnp.int32, L)
        zv = jnp.zeros((L,), jnp.float32)

        def ids_copies(ci, par):
            s0 = ci * SPC
            return [
                pltpu.make_async_copy(
                    idsT_hbm.at[s0 + j, pl.ds(bw, BB)],
                    ids_v.at[par, pl.ds(j * BB, BB)], sem_ids[par])
                for j in range(SPC)
            ]

        def gather_of(par):
            return pltpu.make_async_copy(
                table_hbm.at[ids_v.at[par]], rows[par], sem_g[par])

        def xt_copy(s_abs, k_):
            return pltpu.make_async_copy(
                xts[k_], out_hbm.at[s_abs, :, wid], sem_x[k_])

        def compute(ci, par):
            rv = rows[par]

            def sp_body(sp, _):
                for k_ in range(2):
                    si = sp * 2 + k_
                    s_abs = ci * SPC + si
                    xb = xts[k_]

                    @pl.when((ci > 0) | (sp > 0))
                    def _():
                        xt_copy(s_abs, k_).wait()

                    mrs = []
                    for j in range(NGS):
                        tokv = si * BB + j * L + lanes
                        jl = j * L + lanes

                        @plsc.parallel_loop(0, E, step=2, unroll=8,
                                            carry=(zv, zv, zv, zv, lanes))
                        def aloop(d, accs):
                            a1, a2, b1, b2, col = accs
                            col2 = (col + 1) & (E - 1)
                            x = (plsc.load_gather(rv, [tokv, col])
                                 + posf_v[s_abs, pl.ds(d, L)])
                            y = (plsc.load_gather(rv, [tokv, col2])
                                 + posf_v[s_abs, pl.ds(d + 1, L)])
                            plsc.store_scatter(xb, [col >> 3, col & 7, jl], x)
                            plsc.store_scatter(xb, [col2 >> 3, col2 & 7, jl], y)
                            return (a1 + x, a2 + x * x, b1 + y, b2 + y * y,
                                    (col + 2) & (E - 1))

                        a1, a2, b1, b2, _ = aloop
                        mean = (a1 + b1) * (1.0 / E)
                        var = (a2 + b2) * (1.0 / E) - mean * mean
                        mrs.append((mean, _rsqrt_vec(var + LN_EPS)))

                    @plsc.parallel_loop(0, E, unroll=8)
                    def cloop(d):
                        eb = d // 8
                        es = d % 8
                        gt = gbT_v[0, d]
                        bt = gbT_v[1, d]
                        for j in range(NGS):
                            xv = xb[eb, es, pl.ds(j * L, L)]
                            m, r = mrs[j]
                            xb[eb, es, pl.ds(j * L, L)] = (xv - m) * r * gt + bt

                    xt_copy(s_abs, k_).start()
                return 0

            lax.fori_loop(0, SPC // 2, sp_body, 0)

        for c in ids_copies(0, 0):
            c.start()
        for c in ids_copies(0, 0):
            c.wait()
        gather_of(0).start()
        for c in ids_copies(1, 1):
            c.start()

        def pair_body(cp, _):
            for par in range(2):
                ci = cp * 2 + par

                @pl.when(ci + 1 < NCHUNK)
                def _():
                    for c in ids_copies(ci + 1, 1 - par):
                        c.wait()
                    gather_of(1 - par).start()

                gather_of(par).wait()
                compute(ci, par)

                @pl.when(ci + 2 < NCHUNK)
                def _():
                    for c in ids_copies(ci + 2, par):
                        c.start()
            return 0

        lax.fori_loop(0, NCHUNK // 2, pair_body, 0)
        xt_copy(S - 2, 0).wait()
        xt_copy(S - 1, 1).wait()

    return k


def kernel(input_ids, token_type_ids, table, gamma, beta):
    B, S = input_ids.shape
    V, E = table.shape
    idsT = input_ids.T.astype(jnp.int32)
    pos = _posenc_np(S, E)
    posf = jnp.asarray(np.concatenate([pos, pos[:, :L]], axis=1))
    gbT = jnp.stack([jnp.tile(gamma[:, None], (1, L)),
                     jnp.tile(beta[:, None], (1, L))])
    o5 = _build(B, S, E)(idsT, table, posf, gbT)
    return jnp.transpose(o5, (2, 4, 0, 1, 3)).reshape(B, S, E)


# speedup vs baseline: 1.0181x; 1.0181x over previous
import functools

import numpy as np
import jax
import jax.numpy as jnp
from jax import lax
from jax.experimental import pallas as pl
from jax.experimental.pallas import tpu as pltpu
from jax.experimental.pallas import tpu_sc as plsc

LN_EPS = 1e-3
L = 16
SPC = 4
BB = 128


def _posenc_np(max_len, d):
    pos = np.arange(max_len)[:, None]
    i = np.arange(d)[None, :]
    ang = pos * (1.0 / np.power(10000, 2 * (i // 2) / np.float32(d)))
    ang[:, 0::2] = np.sin(ang[:, 0::2])
    ang[:, 1::2] = np.cos(ang[:, 1::2])
    return ang.astype(np.float32)


def _rsqrt_vec(v):
    i = lax.bitcast_convert_type(v, jnp.int32)
    i = jnp.int32(0x5F3759DF) - lax.shift_right_logical(i, jnp.int32(1))
    y = lax.bitcast_convert_type(i, jnp.float32)
    for _ in range(2):
        y = y * (1.5 - 0.5 * v * y * y)
    return y


@functools.lru_cache(maxsize=None)
def _build(B, S, E):
    info = plsc.get_sparse_core_info()
    NC, NS = info.num_cores, info.num_subcores
    NW = NC * NS
    assert B % (NW * BB) == 0 and B // NW == BB
    CHUNK = SPC * BB
    NCHUNK = S // SPC
    assert NCHUNK % 2 == 0 and S % SPC == 0
    NGS = BB // L
    mesh = plsc.VectorSubcoreMesh(core_axis_name="c", subcore_axis_name="s")

    @functools.partial(
        pl.kernel,
        mesh=mesh,
        compiler_params=pltpu.CompilerParams(
            use_tc_tiling_on_sc=False, needs_layout_passes=False),
        out_type=jax.ShapeDtypeStruct((S, E // 8, B // BB, 8, BB),
                                      jnp.float32),
        scratch_types=[
            pltpu.VMEM((2, CHUNK), jnp.int32),
            pltpu.VMEM((CHUNK, E), jnp.float32),
            pltpu.VMEM((CHUNK, E), jnp.float32),
            pltpu.VMEM((E // 8, 8, BB), jnp.float32),
            pltpu.VMEM((E // 8, 8, BB), jnp.float32),
            pltpu.VMEM((S * E,), jnp.float32),
            pltpu.VMEM((2, E, L), jnp.float32),
            [pltpu.SemaphoreType.DMA] * 6,
        ],
    )
    def k(idsT_hbm, table_hbm, posf_hbm, gbT_hbm, out_hbm,
          ids_v, rows0, rows1, xt0, xt1, posf_v, gbT_v, sems):
        cid = lax.axis_index("c")
        sid = lax.axis_index("s")
        wid = sid * NC + cid
        bw = wid * BB
        rows = [rows0, rows1]
        xts = [xt0, xt1]
        sem_ids = [sems[0], sems[1]]
        sem_g = [sems[2], sems[3]]
        sem_x = [sems[4], sems[5]]

        pltpu.sync_copy(posf_hbm, posf_v)
        pltpu.sync_copy(gbT_hbm, gbT_v)
        lanes = lax.iota(jnp.int32, L)
        zv = jnp.zeros((L,), jnp.float32)

        def ids_copies(ci, par):
            s0 = ci * SPC
            return [
                pltpu.make_async_copy(
                    idsT_hbm.at[s0 + j, pl.ds(bw, BB)],
                    ids_v.at[par, pl.ds(j * BB, BB)], sem_ids[par])
                for j in range(SPC)
            ]

        def gather_of(par):
            return pltpu.make_async_copy(
                table_hbm.at[ids_v.at[par]], rows[par], sem_g[par])

        def xt_copy(s_abs, k_):
            return pltpu.make_async_copy(
                xts[k_], out_hbm.at[s_abs, :, wid], sem_x[k_])

        def compute(ci, par):
            rv = rows[par]

            def sp_body(sp, _):
                for k_ in range(2):
                    si = sp * 2 + k_
                    s_abs = ci * SPC + si
                    xb = xts[k_]

                    @pl.when((ci > 0) | (sp > 0))
                    def _():
                        xt_copy(s_abs, k_).wait()

                    pbase = jnp.full((L,), s_abs * E, jnp.int32)
                    mrs = []
                    for j in range(NGS):
                        tokv = si * BB + j * L + lanes
                        jl = j * L + lanes

                        @plsc.parallel_loop(0, E, step=2, unroll=8,
                                            carry=(zv, zv, zv, zv, lanes))
                        def aloop(d, accs):
                            a1, a2, b1, b2, col = accs
                            col2 = (col + 1) & (E - 1)
                            x = (plsc.load_gather(rv, [tokv, col])
                                 + plsc.load_gather(posf_v, [pbase + col]))
                            y = (plsc.load_gather(rv, [tokv, col2])
                                 + plsc.load_gather(posf_v, [pbase + col2]))
                            plsc.store_scatter(xb, [col >> 3, col & 7, jl], x)
                            plsc.store_scatter(xb, [col2 >> 3, col2 & 7, jl], y)
                            return (a1 + x, a2 + x * x, b1 + y, b2 + y * y,
                                    (col + 2) & (E - 1))

                        a1, a2, b1, b2, _ = aloop
                        mean = (a1 + b1) * (1.0 / E)
                        var = (a2 + b2) * (1.0 / E) - mean * mean
                        mrs.append((mean, _rsqrt_vec(var + LN_EPS)))

                    @plsc.parallel_loop(0, E, unroll=8)
                    def cloop(d):
                        eb = d // 8
                        es = d % 8
                        gt = gbT_v[0, d]
                        bt = gbT_v[1, d]
                        for j in range(NGS):
                            xv = xb[eb, es, pl.ds(j * L, L)]
                            m, r = mrs[j]
                            xb[eb, es, pl.ds(j * L, L)] = (xv - m) * r * gt + bt

                    xt_copy(s_abs, k_).start()
                return 0

            lax.fori_loop(0, SPC // 2, sp_body, 0)

        for c in ids_copies(0, 0):
            c.start()
        for c in ids_copies(0, 0):
            c.wait()
        gather_of(0).start()
        for c in ids_copies(1, 1):
            c.start()

        def pair_body(cp, _):
            for par in range(2):
                ci = cp * 2 + par

                @pl.when(ci + 1 < NCHUNK)
                def _():
                    for c in ids_copies(ci + 1, 1 - par):
                        c.wait()
                    gather_of(1 - par).start()

                gather_of(par).wait()
                compute(ci, par)

                @pl.when(ci + 2 < NCHUNK)
                def _():
                    for c in ids_copies(ci + 2, par):
                        c.start()
            return 0

        lax.fori_loop(0, NCHUNK // 2, pair_body, 0)
        xt_copy(S - 2, 0).wait()
        xt_copy(S - 1, 1).wait()

    return k


def kernel(input_ids, token_type_ids, table, gamma, beta):
    B, S = input_ids.shape
    V, E = table.shape
    idsT = input_ids.T.astype(jnp.int32)
    posf = jnp.asarray(_posenc_np(S, E).reshape(-1))
    gbT = jnp.stack([jnp.tile(gamma[:, None], (1, L)),
                     jnp.tile(beta[:, None], (1, L))])
    o5 = _build(B, S, E)(idsT, table, posf, gbT)
    return jnp.transpose(o5, (2, 4, 0, 1, 3)).reshape(B, S, E)


# speedup vs baseline: 1.1741x; 1.1533x over previous
import functools

import numpy as np
import jax
import jax.numpy as jnp
from jax import lax
from jax.experimental import pallas as pl
from jax.experimental.pallas import tpu as pltpu
from jax.experimental.pallas import tpu_sc as plsc

LN_EPS = 1e-3
L = 16
SPC = 4
BB = 128


def _posenc_np(max_len, d):
    pos = np.arange(max_len)[:, None]
    i = np.arange(d)[None, :]
    ang = pos * (1.0 / np.power(10000, 2 * (i // 2) / np.float32(d)))
    ang[:, 0::2] = np.sin(ang[:, 0::2])
    ang[:, 1::2] = np.cos(ang[:, 1::2])
    return ang.astype(np.float32)


def _rsqrt_vec(v):
    i = lax.bitcast_convert_type(v, jnp.int32)
    i = jnp.int32(0x5F3759DF) - lax.shift_right_logical(i, jnp.int32(1))
    y = lax.bitcast_convert_type(i, jnp.float32)
    for _ in range(2):
        y = y * (1.5 - 0.5 * v * y * y)
    return y


@functools.lru_cache(maxsize=None)
def _build(B, S, E):
    info = plsc.get_sparse_core_info()
    NC, NS = info.num_cores, info.num_subcores
    NW = NC * NS
    assert B % (NW * BB) == 0 and B // NW == BB
    CHUNK = SPC * BB
    NCHUNK = S // SPC
    assert NCHUNK % 2 == 0 and S % SPC == 0
    NGS = BB // L
    mesh = plsc.VectorSubcoreMesh(core_axis_name="c", subcore_axis_name="s")

    @functools.partial(
        pl.kernel,
        mesh=mesh,
        compiler_params=pltpu.CompilerParams(
            use_tc_tiling_on_sc=False, needs_layout_passes=False),
        out_type=jax.ShapeDtypeStruct((S, E // 8, B // BB, 8, BB),
                                      jnp.float32),
        scratch_types=[
            pltpu.VMEM((2, CHUNK), jnp.int32),
            pltpu.VMEM((CHUNK, E), jnp.float32),
            pltpu.VMEM((CHUNK, E), jnp.float32),
            pltpu.VMEM((E // 8, 8, BB), jnp.float32),
            pltpu.VMEM((E // 8, 8, BB), jnp.float32),
            pltpu.VMEM((S * E,), jnp.float32),
            pltpu.VMEM((2, E, L), jnp.float32),
            [pltpu.SemaphoreType.DMA] * 6,
        ],
    )
    def k(idsT_hbm, table_hbm, posf_hbm, gbT_hbm, out_hbm,
          ids_v, rows0, rows1, xt0, xt1, posf_v, gbT_v, sems):
        cid = lax.axis_index("c")
        sid = lax.axis_index("s")
        wid = sid * NC + cid
        bw = wid * BB
        rows = [rows0, rows1]
        xts = [xt0, xt1]
        sem_ids = [sems[0], sems[1]]
        sem_g = [sems[2], sems[3]]
        sem_x = [sems[4], sems[5]]

        pltpu.sync_copy(posf_hbm, posf_v)
        pltpu.sync_copy(gbT_hbm, gbT_v)
        lanes = lax.iota(jnp.int32, L)
        zv = jnp.zeros((L,), jnp.float32)

        def ids_copies(ci, par):
            s0 = ci * SPC
            return [
                pltpu.make_async_copy(
                    idsT_hbm.at[s0 + j, pl.ds(bw, BB)],
                    ids_v.at[par, pl.ds(j * BB, BB)], sem_ids[par])
                for j in range(SPC)
            ]

        def gather_of(par):
            return pltpu.make_async_copy(
                table_hbm.at[ids_v.at[par]], rows[par], sem_g[par])

        def xt_copy(s_abs, k_):
            return pltpu.make_async_copy(
                xts[k_], out_hbm.at[s_abs, :, wid], sem_x[k_])

        def compute(ci, par):
            rv = rows[par]

            def sp_body(sp, _):
                for k_ in range(2):
                    si = sp * 2 + k_
                    s_abs = ci * SPC + si
                    xb = xts[k_]

                    @pl.when((ci > 0) | (sp > 0))
                    def _():
                        xt_copy(s_abs, k_).wait()

                    pbase = jnp.full((L,), s_abs * E, jnp.int32)
                    mrs = []
                    for j in range(NGS):
                        tokv = si * BB + j * L + lanes
                        jl = j * L + lanes

                        @plsc.parallel_loop(0, E, step=2, unroll=8,
                                            carry=(zv, zv, zv, zv, lanes))
                        def aloop(d, accs):
                            a1, a2, b1, b2, col = accs
                            col2 = (col + 1) & (E - 1)
                            x = (plsc.load_gather(rv, [tokv, col])
                                 + plsc.load_gather(posf_v, [pbase + col]))
                            y = (plsc.load_gather(rv, [tokv, col2])
                                 + plsc.load_gather(posf_v, [pbase + col2]))
                            plsc.store_scatter(xb, [col >> 3, col & 7, jl], x)
                            plsc.store_scatter(xb, [col2 >> 3, col2 & 7, jl], y)
                            return (a1 + x, a2 + x * x, b1 + y, b2 + y * y,
                                    (col + 2) & (E - 1))

                        a1, a2, b1, b2, _ = aloop
                        mean = (a1 + b1) * (1.0 / E)
                        var = (a2 + b2) * (1.0 / E) - mean * mean
                        mrs.append((mean, _rsqrt_vec(var + LN_EPS)))

                    @plsc.parallel_loop(0, E, unroll=4)
                    def cloop(d):
                        eb = d // 8
                        es = d % 8
                        gt = gbT_v[0, d]
                        bt = gbT_v[1, d]
                        for j in range(NGS):
                            xv = xb[eb, es, pl.ds(j * L, L)]
                            m, r = mrs[j]
                            xb[eb, es, pl.ds(j * L, L)] = (xv - m) * r * gt + bt

                    xt_copy(s_abs, k_).start()
                return 0

            lax.fori_loop(0, SPC // 2, sp_body, 0)

        for c in ids_copies(0, 0):
            c.start()
        for c in ids_copies(0, 0):
            c.wait()
        gather_of(0).start()
        for c in ids_copies(1, 1):
            c.start()

        def pair_body(cp, _):
            for par in range(2):
                ci = cp * 2 + par

                @pl.when(ci + 1 < NCHUNK)
                def _():
                    for c in ids_copies(ci + 1, 1 - par):
                        c.wait()
                    gather_of(1 - par).start()

                gather_of(par).wait()
                compute(ci, par)

                @pl.when(ci + 2 < NCHUNK)
                def _():
                    for c in ids_copies(ci + 2, par):
                        c.start()
            return 0

        lax.fori_loop(0, NCHUNK // 2, pair_body, 0)
        xt_copy(S - 2, 0).wait()
        xt_copy(S - 1, 1).wait()

    return k


def kernel(input_ids, token_type_ids, table, gamma, beta):
    B, S = input_ids.shape
    V, E = table.shape
    idsT = input_ids.T.astype(jnp.int32)
    posf = jnp.asarray(_posenc_np(S, E).reshape(-1))
    gbT = jnp.stack([jnp.tile(gamma[:, None], (1, L)),
                     jnp.tile(beta[:, None], (1, L))])
    o5 = _build(B, S, E)(idsT, table, posf, gbT)
    return jnp.transpose(o5, (2, 4, 0, 1, 3)).reshape(B, S, E)
